# baseline (speedup 1.0000x reference)
import functools

import jax
import jax.numpy as jnp
from jax import lax
from jax.experimental import pallas as pl
from jax.experimental.pallas import tpu as pltpu
from jax.experimental.pallas import tpu_sc as plsc

_B, _D = 16384, 128
_T = 1000
_NC, _NS, _L = 2, 16, 16
_NW = _NC * _NS
_BPW = _B // _NW
_R = 128
_NCHUNK = _BPW // _R


def _fused(x_start, x_noise, timesteps, table1, table2):
    mesh = plsc.VectorSubcoreMesh(core_axis_name="c", subcore_axis_name="s")

    @functools.partial(
        pl.kernel,
        out_type=jax.ShapeDtypeStruct((_B, _D), jnp.float32),
        mesh=mesh,
        compiler_params=pltpu.CompilerParams(needs_layout_passes=False),
        scratch_types=[
            pltpu.VMEM((_T,), jnp.float32),
            pltpu.VMEM((_T,), jnp.float32),
            pltpu.VMEM((_BPW,), jnp.int32),
            pltpu.VMEM((_BPW,), jnp.float32),
            pltpu.VMEM((_BPW,), jnp.float32),
            pltpu.VMEM((_R, _D), jnp.float32),
            pltpu.VMEM((_R, _D), jnp.float32),
            pltpu.VMEM((_R, _D), jnp.float32),
        ],
    )
    def fused_kernel(xs_hbm, xn_hbm, ts_hbm, t1_hbm, t2_hbm, out_hbm,
                     t1_v, t2_v, idx_v, s1_v, s2_v, xs_v, xn_v, o_v):
        wid = lax.axis_index("s") * _NC + lax.axis_index("c")
        base = wid * _BPW
        pltpu.sync_copy(t1_hbm, t1_v)
        pltpu.sync_copy(t2_hbm, t2_v)
        pltpu.sync_copy(ts_hbm.at[pl.ds(base, _BPW)], idx_v)

        def gbody(i, carry):
            sl = pl.ds(i * _L, _L)
            idx = idx_v[sl]
            s1_v[sl] = plsc.load_gather(t1_v, [idx])
            s2_v[sl] = plsc.load_gather(t2_v, [idx])
            return carry

        lax.fori_loop(0, _BPW // _L, gbody, 0, unroll=8)

        def chunk(c, carry):
            row0 = base + c * _R
            pltpu.sync_copy(xs_hbm.at[pl.ds(row0, _R), :], xs_v)
            pltpu.sync_copy(xn_hbm.at[pl.ds(row0, _R), :], xn_v)

            def rowfn(r, carry2):
                gr = c * _R + r
                vr = jnp.full((_L,), gr, jnp.int32)
                a = plsc.load_gather(s1_v, [vr])
                b = plsc.load_gather(s2_v, [vr])
                for j in range(_D // _L):
                    sl = pl.ds(j * _L, _L)
                    o_v[r, sl] = a * xs_v[r, sl] + b * xn_v[r, sl]
                return carry2

            lax.fori_loop(0, _R, rowfn, 0, unroll=2)
            pltpu.sync_copy(o_v, out_hbm.at[pl.ds(row0, _R), :])
            return carry

        lax.fori_loop(0, _NCHUNK, chunk, 0)

    return fused_kernel(x_start, x_noise, timesteps, table1, table2)


def kernel(x_start, x_noise, timesteps, sqrt_alphas_cumprod,
           sqrt_one_minus_alphas_cumprod):
    return _fused(x_start, x_noise, timesteps, sqrt_alphas_cumprod,
                  sqrt_one_minus_alphas_cumprod)


# speedup vs baseline: 1.5798x; 1.5798x over previous
import functools

import jax
import jax.numpy as jnp
from jax import lax
from jax.experimental import pallas as pl
from jax.experimental.pallas import tpu as pltpu
from jax.experimental.pallas import tpu_sc as plsc

_B, _D = 16384, 128
_T = 1000
_NC, _NS, _L = 2, 16, 16
_NW = _NC * _NS
_BPW = _B // _NW


def _gather_coeffs(table1, table2, timesteps):
    mesh = plsc.VectorSubcoreMesh(core_axis_name="c", subcore_axis_name="s")

    @functools.partial(
        pl.kernel,
        out_type=(
            jax.ShapeDtypeStruct((_B,), jnp.float32),
            jax.ShapeDtypeStruct((_B,), jnp.float32),
        ),
        mesh=mesh,
        compiler_params=pltpu.CompilerParams(needs_layout_passes=False),
        scratch_types=[
            pltpu.VMEM((_T,), jnp.float32),
            pltpu.VMEM((_T,), jnp.float32),
            pltpu.VMEM((_BPW,), jnp.int32),
            pltpu.VMEM((_BPW,), jnp.float32),
            pltpu.VMEM((_BPW,), jnp.float32),
        ],
    )
    def gather_kernel(t1_hbm, t2_hbm, ts_hbm, s1_hbm, s2_hbm,
                      t1_v, t2_v, idx_v, s1_v, s2_v):
        wid = lax.axis_index("s") * _NC + lax.axis_index("c")
        base = wid * _BPW
        pltpu.sync_copy(t1_hbm, t1_v)
        pltpu.sync_copy(t2_hbm, t2_v)
        pltpu.sync_copy(ts_hbm.at[pl.ds(base, _BPW)], idx_v)

        def body(i, carry):
            sl = pl.ds(i * _L, _L)
            idx = idx_v[sl]
            s1_v[sl] = plsc.load_gather(t1_v, [idx])
            s2_v[sl] = plsc.load_gather(t2_v, [idx])
            return carry

        lax.fori_loop(0, _BPW // _L, body, 0, unroll=8)

        pltpu.sync_copy(s1_v, s1_hbm.at[pl.ds(base, _BPW)])
        pltpu.sync_copy(s2_v, s2_hbm.at[pl.ds(base, _BPW)])

    return gather_kernel(table1, table2, timesteps)


def _blend(s1, s2, x_start, x_noise):
    bs = 1024

    def body(s1_ref, s2_ref, xs_ref, xn_ref, o_ref):
        c1 = s1_ref[...].reshape(bs, 1)
        c2 = s2_ref[...].reshape(bs, 1)
        o_ref[...] = c1 * xs_ref[...] + c2 * xn_ref[...]

    return pl.pallas_call(
        body,
        grid=(_B // bs,),
        in_specs=[
            pl.BlockSpec((bs,), lambda i: (i,)),
            pl.BlockSpec((bs,), lambda i: (i,)),
            pl.BlockSpec((bs, _D), lambda i: (i, 0)),
            pl.BlockSpec((bs, _D), lambda i: (i, 0)),
        ],
        out_specs=pl.BlockSpec((bs, _D), lambda i: (i, 0)),
        out_shape=jax.ShapeDtypeStruct((_B, _D), jnp.float32),
    )(s1, s2, x_start, x_noise)


def kernel(x_start, x_noise, timesteps, sqrt_alphas_cumprod,
           sqrt_one_minus_alphas_cumprod):
    s1, s2 = _gather_coeffs(sqrt_alphas_cumprod,
                            sqrt_one_minus_alphas_cumprod, timesteps)
    return _blend(s1, s2, x_start, x_noise)


# speedup vs baseline: 1.7763x; 1.1243x over previous
import functools

import jax
import jax.numpy as jnp
from jax import lax
from jax.experimental import pallas as pl
from jax.experimental.pallas import tpu as pltpu
from jax.experimental.pallas import tpu_sc as plsc

_B, _D = 16384, 128
_T = 1000
_NC, _NS, _L = 2, 16, 16
_NW = _NC * _NS
_BPW = _B // _NW


def _gather_coeffs(table1, table2, timesteps):
    mesh = plsc.VectorSubcoreMesh(core_axis_name="c", subcore_axis_name="s")

    @functools.partial(
        pl.kernel,
        out_type=(
            jax.ShapeDtypeStruct((_B,), jnp.float32),
            jax.ShapeDtypeStruct((_B,), jnp.float32),
        ),
        mesh=mesh,
        compiler_params=pltpu.CompilerParams(needs_layout_passes=False),
        scratch_types=[
            pltpu.VMEM((_T,), jnp.float32),
            pltpu.VMEM((_T,), jnp.float32),
            pltpu.VMEM((_BPW,), jnp.int32),
            pltpu.VMEM((_BPW,), jnp.float32),
            pltpu.VMEM((_BPW,), jnp.float32),
        ],
    )
    def gather_kernel(t1_hbm, t2_hbm, ts_hbm, s1_hbm, s2_hbm,
                      t1_v, t2_v, idx_v, s1_v, s2_v):
        wid = lax.axis_index("s") * _NC + lax.axis_index("c")
        base = wid * _BPW
        pltpu.sync_copy(t1_hbm, t1_v)
        pltpu.sync_copy(t2_hbm, t2_v)
        pltpu.sync_copy(ts_hbm.at[pl.ds(base, _BPW)], idx_v)

        def body(i, carry):
            sl = pl.ds(i * _L, _L)
            idx = idx_v[sl]
            s1_v[sl] = plsc.load_gather(t1_v, [idx])
            s2_v[sl] = plsc.load_gather(t2_v, [idx])
            return carry

        lax.fori_loop(0, _BPW // _L, body, 0, unroll=8)

        pltpu.sync_copy(s1_v, s1_hbm.at[pl.ds(base, _BPW)])
        pltpu.sync_copy(s2_v, s2_hbm.at[pl.ds(base, _BPW)])

    return gather_kernel(table1, table2, timesteps)


def _blend(s1, s2, x_start, x_noise):
    bs = 2048

    def body(s1_ref, s2_ref, xs_ref, xn_ref, o_ref):
        c1 = s1_ref[...].reshape(bs, 1)
        c2 = s2_ref[...].reshape(bs, 1)
        o_ref[...] = c1 * xs_ref[...] + c2 * xn_ref[...]

    return pl.pallas_call(
        body,
        grid=(_B // bs,),
        in_specs=[
            pl.BlockSpec((bs,), lambda i: (i,)),
            pl.BlockSpec((bs,), lambda i: (i,)),
            pl.BlockSpec((bs, _D), lambda i: (i, 0)),
            pl.BlockSpec((bs, _D), lambda i: (i, 0)),
        ],
        out_specs=pl.BlockSpec((bs, _D), lambda i: (i, 0)),
        out_shape=jax.ShapeDtypeStruct((_B, _D), jnp.float32),
    )(s1, s2, x_start, x_noise)


def kernel(x_start, x_noise, timesteps, sqrt_alphas_cumprod,
           sqrt_one_minus_alphas_cumprod):
    s1, s2 = _gather_coeffs(sqrt_alphas_cumprod,
                            sqrt_one_minus_alphas_cumprod, timesteps)
    return _blend(s1, s2, x_start, x_noise)


# speedup vs baseline: 1.8638x; 1.0493x over previous
import functools

import jax
import jax.numpy as jnp
from jax import lax
from jax.experimental import pallas as pl
from jax.experimental.pallas import tpu as pltpu
from jax.experimental.pallas import tpu_sc as plsc

_B, _D = 16384, 128
_T = 1000
_NC, _NS, _L = 2, 16, 16
_NW = _NC * _NS
_BPW = _B // _NW


def _gather_coeffs(table1, table2, timesteps):
    mesh = plsc.VectorSubcoreMesh(core_axis_name="c", subcore_axis_name="s")

    @functools.partial(
        pl.kernel,
        out_type=(
            jax.ShapeDtypeStruct((_B,), jnp.float32),
            jax.ShapeDtypeStruct((_B,), jnp.float32),
        ),
        mesh=mesh,
        compiler_params=pltpu.CompilerParams(needs_layout_passes=False),
        scratch_types=[
            pltpu.VMEM((_T,), jnp.float32),
            pltpu.VMEM((_T,), jnp.float32),
            pltpu.VMEM((_BPW,), jnp.int32),
            pltpu.VMEM((_BPW,), jnp.float32),
            pltpu.VMEM((_BPW,), jnp.float32),
        ],
    )
    def gather_kernel(t1_hbm, t2_hbm, ts_hbm, s1_hbm, s2_hbm,
                      t1_v, t2_v, idx_v, s1_v, s2_v):
        wid = lax.axis_index("s") * _NC + lax.axis_index("c")
        base = wid * _BPW
        pltpu.sync_copy(t1_hbm, t1_v)
        pltpu.sync_copy(t2_hbm, t2_v)
        pltpu.sync_copy(ts_hbm.at[pl.ds(base, _BPW)], idx_v)

        def body(i, carry):
            sl = pl.ds(i * _L, _L)
            idx = idx_v[sl]
            s1_v[sl] = plsc.load_gather(t1_v, [idx])
            s2_v[sl] = plsc.load_gather(t2_v, [idx])
            return carry

        lax.fori_loop(0, _BPW // _L, body, 0, unroll=8)

        pltpu.sync_copy(s1_v, s1_hbm.at[pl.ds(base, _BPW)])
        pltpu.sync_copy(s2_v, s2_hbm.at[pl.ds(base, _BPW)])

    return gather_kernel(table1, table2, timesteps)


def _blend(s1, s2, x_start, x_noise):
    bs = 4096

    def body(s1_ref, s2_ref, xs_ref, xn_ref, o_ref):
        c1 = s1_ref[...].reshape(bs, 1)
        c2 = s2_ref[...].reshape(bs, 1)
        o_ref[...] = c1 * xs_ref[...] + c2 * xn_ref[...]

    return pl.pallas_call(
        body,
        grid=(_B // bs,),
        in_specs=[
            pl.BlockSpec((bs,), lambda i: (i,)),
            pl.BlockSpec((bs,), lambda i: (i,)),
            pl.BlockSpec((bs, _D), lambda i: (i, 0)),
            pl.BlockSpec((bs, _D), lambda i: (i, 0)),
        ],
        out_specs=pl.BlockSpec((bs, _D), lambda i: (i, 0)),
        out_shape=jax.ShapeDtypeStruct((_B, _D), jnp.float32),
    )(s1, s2, x_start, x_noise)


def kernel(x_start, x_noise, timesteps, sqrt_alphas_cumprod,
           sqrt_one_minus_alphas_cumprod):
    s1, s2 = _gather_coeffs(sqrt_alphas_cumprod,
                            sqrt_one_minus_alphas_cumprod, timesteps)
    return _blend(s1, s2, x_start, x_noise)


# speedup vs baseline: 1.9023x; 1.0207x over previous
import functools

import jax
import jax.numpy as jnp
from jax import lax
from jax.experimental import pallas as pl
from jax.experimental.pallas import tpu as pltpu
from jax.experimental.pallas import tpu_sc as plsc

_B, _D = 16384, 128
_T = 1000
_NC, _NS, _L = 2, 16, 16
_NW = _NC * _NS
_BPW = _B // _NW


def _gather_coeffs(table1, table2, timesteps):
    mesh = plsc.VectorSubcoreMesh(core_axis_name="c", subcore_axis_name="s")

    @functools.partial(
        pl.kernel,
        out_type=(
            jax.ShapeDtypeStruct((_B,), jnp.float32),
            jax.ShapeDtypeStruct((_B,), jnp.float32),
        ),
        mesh=mesh,
        compiler_params=pltpu.CompilerParams(needs_layout_passes=False),
        scratch_types=[
            pltpu.VMEM((_T,), jnp.float32),
            pltpu.VMEM((_T,), jnp.float32),
            pltpu.VMEM((_BPW,), jnp.int32),
            pltpu.VMEM((_BPW,), jnp.float32),
            pltpu.VMEM((_BPW,), jnp.float32),
        ],
    )
    def gather_kernel(t1_hbm, t2_hbm, ts_hbm, s1_hbm, s2_hbm,
                      t1_v, t2_v, idx_v, s1_v, s2_v):
        wid = lax.axis_index("s") * _NC + lax.axis_index("c")
        base = wid * _BPW
        pltpu.sync_copy(t1_hbm, t1_v)
        pltpu.sync_copy(t2_hbm, t2_v)
        pltpu.sync_copy(ts_hbm.at[pl.ds(base, _BPW)], idx_v)

        def body(i, carry):
            sl = pl.ds(i * _L, _L)
            idx = idx_v[sl]
            s1_v[sl] = plsc.load_gather(t1_v, [idx])
            s2_v[sl] = plsc.load_gather(t2_v, [idx])
            return carry

        lax.fori_loop(0, _BPW // _L, body, 0, unroll=8)

        pltpu.sync_copy(s1_v, s1_hbm.at[pl.ds(base, _BPW)])
        pltpu.sync_copy(s2_v, s2_hbm.at[pl.ds(base, _BPW)])

    return gather_kernel(table1, table2, timesteps)


def _blend(s1, s2, x_start, x_noise):
    bs = 8192

    def body(s1_ref, s2_ref, xs_ref, xn_ref, o_ref):
        c1 = s1_ref[...].reshape(bs, 1)
        c2 = s2_ref[...].reshape(bs, 1)
        o_ref[...] = c1 * xs_ref[...] + c2 * xn_ref[...]

    return pl.pallas_call(
        body,
        grid=(_B // bs,),
        in_specs=[
            pl.BlockSpec((bs,), lambda i: (i,)),
            pl.BlockSpec((bs,), lambda i: (i,)),
            pl.BlockSpec((bs, _D), lambda i: (i, 0)),
            pl.BlockSpec((bs, _D), lambda i: (i, 0)),
        ],
        out_specs=pl.BlockSpec((bs, _D), lambda i: (i, 0)),
        out_shape=jax.ShapeDtypeStruct((_B, _D), jnp.float32),
    )(s1, s2, x_start, x_noise)


def kernel(x_start, x_noise, timesteps, sqrt_alphas_cumprod,
           sqrt_one_minus_alphas_cumprod):
    s1, s2 = _gather_coeffs(sqrt_alphas_cumprod,
                            sqrt_one_minus_alphas_cumprod, timesteps)
    return _blend(s1, s2, x_start, x_noise)


# speedup vs baseline: 1.9840x; 1.0429x over previous
import functools

import jax
import jax.numpy as jnp
from jax import lax
from jax.experimental import pallas as pl
from jax.experimental.pallas import tpu as pltpu
from jax.experimental.pallas import tpu_sc as plsc

_B, _D = 16384, 128
_T = 1000
_NC, _NS, _L = 2, 16, 16
_NW = _NC * _NS
_BPW = _B // _NW


def _gather_coeffs(table1, table2, timesteps):
    mesh = plsc.VectorSubcoreMesh(core_axis_name="c", subcore_axis_name="s")

    @functools.partial(
        pl.kernel,
        out_type=(
            jax.ShapeDtypeStruct((_B,), jnp.float32),
            jax.ShapeDtypeStruct((_B,), jnp.float32),
        ),
        mesh=mesh,
        compiler_params=pltpu.CompilerParams(needs_layout_passes=False),
        scratch_types=[
            pltpu.VMEM((_T,), jnp.float32),
            pltpu.VMEM((_T,), jnp.float32),
            pltpu.VMEM((_BPW,), jnp.int32),
            pltpu.VMEM((_BPW,), jnp.float32),
            pltpu.VMEM((_BPW,), jnp.float32),
            pltpu.SemaphoreType.DMA,
            pltpu.SemaphoreType.DMA,
            pltpu.SemaphoreType.DMA,
        ],
    )
    def gather_kernel(t1_hbm, t2_hbm, ts_hbm, s1_hbm, s2_hbm,
                      t1_v, t2_v, idx_v, s1_v, s2_v, sem1, sem2, sem3):
        wid = lax.axis_index("s") * _NC + lax.axis_index("c")
        base = wid * _BPW
        c1 = pltpu.async_copy(t1_hbm, t1_v, sem1)
        c2 = pltpu.async_copy(t2_hbm, t2_v, sem2)
        c3 = pltpu.async_copy(ts_hbm.at[pl.ds(base, _BPW)], idx_v, sem3)
        c1.wait()
        c2.wait()
        c3.wait()

        def body(i, carry):
            sl = pl.ds(i * _L, _L)
            idx = idx_v[sl]
            s1_v[sl] = plsc.load_gather(t1_v, [idx])
            s2_v[sl] = plsc.load_gather(t2_v, [idx])
            return carry

        lax.fori_loop(0, _BPW // _L, body, 0, unroll=8)

        o1 = pltpu.async_copy(s1_v, s1_hbm.at[pl.ds(base, _BPW)], sem1)
        o2 = pltpu.async_copy(s2_v, s2_hbm.at[pl.ds(base, _BPW)], sem2)
        o1.wait()
        o2.wait()

    return gather_kernel(table1, table2, timesteps)


def _blend(s1, s2, x_start, x_noise):
    bs = 8192

    def body(s1_ref, s2_ref, xs_ref, xn_ref, o_ref):
        c1 = s1_ref[...].reshape(bs, 1)
        c2 = s2_ref[...].reshape(bs, 1)
        o_ref[...] = c1 * xs_ref[...] + c2 * xn_ref[...]

    return pl.pallas_call(
        body,
        grid=(_B // bs,),
        in_specs=[
            pl.BlockSpec((bs,), lambda i: (i,)),
            pl.BlockSpec((bs,), lambda i: (i,)),
            pl.BlockSpec((bs, _D), lambda i: (i, 0)),
            pl.BlockSpec((bs, _D), lambda i: (i, 0)),
        ],
        out_specs=pl.BlockSpec((bs, _D), lambda i: (i, 0)),
        out_shape=jax.ShapeDtypeStruct((_B, _D), jnp.float32),
    )(s1, s2, x_start, x_noise)


def kernel(x_start, x_noise, timesteps, sqrt_alphas_cumprod,
           sqrt_one_minus_alphas_cumprod):
    s1, s2 = _gather_coeffs(sqrt_alphas_cumprod,
                            sqrt_one_minus_alphas_cumprod, timesteps)
    return _blend(s1, s2, x_start, x_noise)


# speedup vs baseline: 1.9892x; 1.0026x over previous
import functools

import jax
import jax.numpy as jnp
from jax import lax
from jax.experimental import pallas as pl
from jax.experimental.pallas import tpu as pltpu
from jax.experimental.pallas import tpu_sc as plsc

_B, _D = 16384, 128
_T = 1000
_NC, _NS, _L = 2, 16, 16
_NW = _NC * _NS
_BPW = _B // _NW


def _gather_coeffs(table1, table2, timesteps):
    mesh = plsc.VectorSubcoreMesh(core_axis_name="c", subcore_axis_name="s")

    @functools.partial(
        pl.kernel,
        out_type=(
            jax.ShapeDtypeStruct((_B,), jnp.float32),
            jax.ShapeDtypeStruct((_B,), jnp.float32),
        ),
        mesh=mesh,
        compiler_params=pltpu.CompilerParams(needs_layout_passes=False, use_tc_tiling_on_sc=True),
        scratch_types=[
            pltpu.VMEM((_T,), jnp.float32),
            pltpu.VMEM((_T,), jnp.float32),
            pltpu.VMEM((_BPW,), jnp.int32),
            pltpu.VMEM((_BPW,), jnp.float32),
            pltpu.VMEM((_BPW,), jnp.float32),
            pltpu.SemaphoreType.DMA,
            pltpu.SemaphoreType.DMA,
            pltpu.SemaphoreType.DMA,
        ],
    )
    def gather_kernel(t1_hbm, t2_hbm, ts_hbm, s1_hbm, s2_hbm,
                      t1_v, t2_v, idx_v, s1_v, s2_v, sem1, sem2, sem3):
        wid = lax.axis_index("s") * _NC + lax.axis_index("c")
        base = wid * _BPW
        c1 = pltpu.async_copy(t1_hbm, t1_v, sem1)
        c2 = pltpu.async_copy(t2_hbm, t2_v, sem2)
        c3 = pltpu.async_copy(ts_hbm.at[pl.ds(base, _BPW)], idx_v, sem3)
        c1.wait()
        c2.wait()
        c3.wait()

        def body(i, carry):
            sl = pl.ds(i * _L, _L)
            idx = idx_v[sl]
            s1_v[sl] = plsc.load_gather(t1_v, [idx])
            s2_v[sl] = plsc.load_gather(t2_v, [idx])
            return carry

        lax.fori_loop(0, _BPW // _L, body, 0, unroll=8)

        o1 = pltpu.async_copy(s1_v, s1_hbm.at[pl.ds(base, _BPW)], sem1)
        o2 = pltpu.async_copy(s2_v, s2_hbm.at[pl.ds(base, _BPW)], sem2)
        o1.wait()
        o2.wait()

    return gather_kernel(table1, table2, timesteps)


def _blend(s1, s2, x_start, x_noise):
    bs = 8192

    def body(s1_ref, s2_ref, xs_ref, xn_ref, o_ref):
        c1 = s1_ref[...].reshape(bs, 1)
        c2 = s2_ref[...].reshape(bs, 1)
        o_ref[...] = c1 * xs_ref[...] + c2 * xn_ref[...]

    return pl.pallas_call(
        body,
        grid=(_B // bs,),
        in_specs=[
            pl.BlockSpec((bs,), lambda i: (i,)),
            pl.BlockSpec((bs,), lambda i: (i,)),
            pl.BlockSpec((bs, _D), lambda i: (i, 0)),
            pl.BlockSpec((bs, _D), lambda i: (i, 0)),
        ],
        out_specs=pl.BlockSpec((bs, _D), lambda i: (i, 0)),
        out_shape=jax.ShapeDtypeStruct((_B, _D), jnp.float32),
    )(s1, s2, x_start, x_noise)


def kernel(x_start, x_noise, timesteps, sqrt_alphas_cumprod,
           sqrt_one_minus_alphas_cumprod):
    s1, s2 = _gather_coeffs(sqrt_alphas_cumprod,
                            sqrt_one_minus_alphas_cumprod, timesteps)
    return _blend(s1, s2, x_start, x_noise)


# speedup vs baseline: 1.9938x; 1.0023x over previous
import functools

import jax
import jax.numpy as jnp
from jax import lax
from jax.experimental import pallas as pl
from jax.experimental.pallas import tpu as pltpu
from jax.experimental.pallas import tpu_sc as plsc

_B, _D = 16384, 128
_T = 1000
_NC, _NS, _L = 2, 16, 16
_NW = _NC * _NS
_BPW = _B // _NW


def _gather_coeffs(table1, table2, timesteps):
    mesh = plsc.VectorSubcoreMesh(core_axis_name="c", subcore_axis_name="s")

    @functools.partial(
        pl.kernel,
        out_type=(
            jax.ShapeDtypeStruct((_B,), jnp.float32),
            jax.ShapeDtypeStruct((_B,), jnp.float32),
        ),
        mesh=mesh,
        compiler_params=pltpu.CompilerParams(needs_layout_passes=False, use_tc_tiling_on_sc=True),
        scratch_types=[
            pltpu.VMEM((_T,), jnp.float32),
            pltpu.VMEM((_T,), jnp.float32),
            pltpu.VMEM((_BPW,), jnp.int32),
            pltpu.VMEM((_BPW,), jnp.float32),
            pltpu.VMEM((_BPW,), jnp.float32),
            pltpu.SemaphoreType.DMA,
            pltpu.SemaphoreType.DMA,
            pltpu.SemaphoreType.DMA,
        ],
    )
    def gather_kernel(t1_hbm, t2_hbm, ts_hbm, s1_hbm, s2_hbm,
                      t1_v, t2_v, idx_v, s1_v, s2_v, sem1, sem2, sem3):
        wid = lax.axis_index("s") * _NC + lax.axis_index("c")
        base = wid * _BPW
        c1 = pltpu.async_copy(t1_hbm, t1_v, sem1)
        c2 = pltpu.async_copy(t2_hbm, t2_v, sem2)
        c3 = pltpu.async_copy(ts_hbm.at[pl.ds(base, _BPW)], idx_v, sem3)
        c1.wait()
        c2.wait()
        c3.wait()

        for i in range(_BPW // _L):
            sl = pl.ds(i * _L, _L)
            idx = idx_v[sl]
            s1_v[sl] = plsc.load_gather(t1_v, [idx])
            s2_v[sl] = plsc.load_gather(t2_v, [idx])

        o1 = pltpu.async_copy(s1_v, s1_hbm.at[pl.ds(base, _BPW)], sem1)
        o2 = pltpu.async_copy(s2_v, s2_hbm.at[pl.ds(base, _BPW)], sem2)
        o1.wait()
        o2.wait()

    return gather_kernel(table1, table2, timesteps)


def _blend(s1, s2, x_start, x_noise):
    bs = 8192

    def body(s1_ref, s2_ref, xs_ref, xn_ref, o_ref):
        c1 = s1_ref[...].reshape(bs, 1)
        c2 = s2_ref[...].reshape(bs, 1)
        o_ref[...] = c1 * xs_ref[...] + c2 * xn_ref[...]

    return pl.pallas_call(
        body,
        grid=(_B // bs,),
        in_specs=[
            pl.BlockSpec((bs,), lambda i: (i,)),
            pl.BlockSpec((bs,), lambda i: (i,)),
            pl.BlockSpec((bs, _D), lambda i: (i, 0)),
            pl.BlockSpec((bs, _D), lambda i: (i, 0)),
        ],
        out_specs=pl.BlockSpec((bs, _D), lambda i: (i, 0)),
        out_shape=jax.ShapeDtypeStruct((_B, _D), jnp.float32),
    )(s1, s2, x_start, x_noise)


def kernel(x_start, x_noise, timesteps, sqrt_alphas_cumprod,
           sqrt_one_minus_alphas_cumprod):
    s1, s2 = _gather_coeffs(sqrt_alphas_cumprod,
                            sqrt_one_minus_alphas_cumprod, timesteps)
    return _blend(s1, s2, x_start, x_noise)
